# baseline (speedup 1.0000x reference)
import jax
import jax.numpy as jnp
from jax.experimental import pallas as pl

N = 4096
D = 128
H1 = 256
H2 = 64
ODIM = 32
K = 10
BR = 512


def _rowsum_prep_kernel(a_ref, x_ref, dinv_ref, xp_ref):
    a = a_ref[...]
    d = jnp.sum(a, axis=1, keepdims=True) + 1.0
    dinv = jax.lax.rsqrt(jnp.maximum(d, 1e-12))
    dinv_ref[...] = dinv
    xp_ref[...] = x_ref[...] * dinv


def _rowsum_prep(A, X):
    return pl.pallas_call(
        _rowsum_prep_kernel,
        grid=(N // BR,),
        in_specs=[
            pl.BlockSpec((BR, N), lambda i: (i, 0)),
            pl.BlockSpec((BR, D), lambda i: (i, 0)),
        ],
        out_specs=[
            pl.BlockSpec((BR, 1), lambda i: (i, 0)),
            pl.BlockSpec((BR, D), lambda i: (i, 0)),
        ],
        out_shape=[
            jax.ShapeDtypeStruct((N, 1), jnp.float32),
            jax.ShapeDtypeStruct((N, D), jnp.float32),
        ],
    )(A, X)


def _layer1_kernel(a_ref, xp_ref, dinv_ref, w1_ref, w2_ref, out_ref):
    i = pl.program_id(0)
    a = a_ref[...]
    xp = xp_ref[...]
    t = jnp.dot(a, xp, preferred_element_type=jnp.float32)
    t = t + xp_ref[pl.ds(i * BR, BR), :]
    dinv = dinv_ref[pl.ds(i * BR, BR), :]
    t = t * dinv
    r = jnp.maximum(jnp.dot(t, w1_ref[...], preferred_element_type=jnp.float32), 0.0)
    out_ref[...] = jnp.dot(r, w2_ref[...], preferred_element_type=jnp.float32) * dinv


def _layer1(A, Xp, dinv, W1, W2):
    return pl.pallas_call(
        _layer1_kernel,
        grid=(N // BR,),
        in_specs=[
            pl.BlockSpec((BR, N), lambda i: (i, 0)),
            pl.BlockSpec((N, D), lambda i: (0, 0)),
            pl.BlockSpec((N, 1), lambda i: (0, 0)),
            pl.BlockSpec((D, H1), lambda i: (0, 0)),
            pl.BlockSpec((H1, H2), lambda i: (0, 0)),
        ],
        out_specs=pl.BlockSpec((BR, H2), lambda i: (i, 0)),
        out_shape=jax.ShapeDtypeStruct((N, H2), jnp.float32),
    )(A, Xp, dinv, W1, W2)


def _layer2_kernel(a_ref, m_ref, dinv_ref, out_ref):
    i = pl.program_id(0)
    a = a_ref[...]
    m = m_ref[...]
    t = jnp.dot(a, m, preferred_element_type=jnp.float32)
    t = t + m_ref[pl.ds(i * BR, BR), :]
    out_ref[...] = t * dinv_ref[pl.ds(i * BR, BR), :]


def _layer2(A, M, dinv):
    return pl.pallas_call(
        _layer2_kernel,
        grid=(N // BR,),
        in_specs=[
            pl.BlockSpec((BR, N), lambda i: (i, 0)),
            pl.BlockSpec((N, H2), lambda i: (0, 0)),
            pl.BlockSpec((N, 1), lambda i: (0, 0)),
        ],
        out_specs=pl.BlockSpec((BR, H2), lambda i: (i, 0)),
        out_shape=jax.ShapeDtypeStruct((N, H2), jnp.float32),
    )(A, M, dinv)


def _epilogue_kernel(hv_ref, hg_ref, wd1_ref, wd2_ref, wf_ref, bf_ref, c_ref,
                     h_ref, q_ref, p_ref, xhat_ref):
    hv = hv_ref[...]
    hg = hg_ref[...]
    r = jnp.maximum(jnp.dot(hv, wd1_ref[...], preferred_element_type=jnp.float32), 0.0)
    xhat_ref[...] = jnp.dot(r, wd2_ref[...], preferred_element_type=jnp.float32)
    wf = wf_ref[...]
    t = (jnp.dot(hv, wf[:H2], preferred_element_type=jnp.float32)
         + jnp.dot(hg, wf[H2:], preferred_element_type=jnp.float32)
         + bf_ref[...])
    h = jnp.tanh(t)
    h_ref[...] = h
    c = c_ref[...]
    cross = jnp.dot(h, c.T, preferred_element_type=jnp.float32)
    dist2 = (jnp.sum(h * h, axis=1, keepdims=True)
             + jnp.sum(c * c, axis=1)[None, :]
             - 2.0 * cross)
    q = 1.0 / (1.0 + dist2)
    q = q / jnp.sum(q, axis=1, keepdims=True)
    q_ref[...] = q
    f = jnp.sum(q, axis=0, keepdims=True)
    p = (q * q) / f
    p_ref[...] = p / jnp.sum(p, axis=1, keepdims=True)


def _epilogue(h_v, h_g, Wd1, Wd2, Wf, bf, centers):
    return pl.pallas_call(
        _epilogue_kernel,
        out_shape=[
            jax.ShapeDtypeStruct((N, ODIM), jnp.float32),
            jax.ShapeDtypeStruct((N, K), jnp.float32),
            jax.ShapeDtypeStruct((N, K), jnp.float32),
            jax.ShapeDtypeStruct((N, D), jnp.float32),
        ],
    )(h_v, h_g, Wd1, Wd2, Wf, bf, centers)


def kernel(X, adj_v, adj_glo, W1_v, W2_v, W1_g, W2_g, Wd1, Wd2, Wf, bf, centers):
    dinv_v, Xp_v = _rowsum_prep(adj_v, X)
    m2_v = _layer1(adj_v, Xp_v, dinv_v, W1_v, W2_v)
    h_v = _layer2(adj_v, m2_v, dinv_v)

    dinv_g, Xp_g = _rowsum_prep(adj_glo, X)
    m2_g = _layer1(adj_glo, Xp_g, dinv_g, W1_g, W2_g)
    h_g = _layer2(adj_glo, m2_g, dinv_g)

    h, q, p, X_hat = _epilogue(h_v, h_g, Wd1, Wd2, Wf, bf.reshape(1, ODIM), centers)
    return (h, q, p, X_hat)


# speedup vs baseline: 1.0329x; 1.0329x over previous
import jax
import jax.numpy as jnp
from jax.experimental import pallas as pl

N = 4096
D = 128
H1 = 256
H2 = 64
ODIM = 32
K = 10
BR = 512


def _rowsum_prep_kernel(a_ref, x_ref, dinv_ref, xp_ref, abf_ref):
    a = a_ref[...]
    d = jnp.sum(a, axis=1, keepdims=True) + 1.0
    dinv = jax.lax.rsqrt(jnp.maximum(d, 1e-12))
    dinv_ref[...] = dinv
    xp_ref[...] = x_ref[...] * dinv
    abf_ref[...] = a.astype(jnp.bfloat16)


def _rowsum_prep(A, X):
    return pl.pallas_call(
        _rowsum_prep_kernel,
        grid=(N // BR,),
        in_specs=[
            pl.BlockSpec((BR, N), lambda i: (i, 0)),
            pl.BlockSpec((BR, D), lambda i: (i, 0)),
        ],
        out_specs=[
            pl.BlockSpec((BR, 1), lambda i: (i, 0)),
            pl.BlockSpec((BR, D), lambda i: (i, 0)),
            pl.BlockSpec((BR, N), lambda i: (i, 0)),
        ],
        out_shape=[
            jax.ShapeDtypeStruct((N, 1), jnp.float32),
            jax.ShapeDtypeStruct((N, D), jnp.float32),
            jax.ShapeDtypeStruct((N, N), jnp.bfloat16),
        ],
    )(A, X)


def _layer1_kernel(a_ref, xp_ref, dinv_ref, w1_ref, w2_ref, out_ref):
    i = pl.program_id(0)
    a = a_ref[...]
    xp = xp_ref[...]
    t = jnp.dot(a, xp.astype(jnp.bfloat16), preferred_element_type=jnp.float32)
    t = t + xp_ref[pl.ds(i * BR, BR), :]
    dinv = dinv_ref[pl.ds(i * BR, BR), :]
    t = t * dinv
    r = jnp.maximum(jnp.dot(t, w1_ref[...], preferred_element_type=jnp.float32), 0.0)
    out_ref[...] = jnp.dot(r, w2_ref[...], preferred_element_type=jnp.float32) * dinv


def _layer1(A, Xp, dinv, W1, W2):
    return pl.pallas_call(
        _layer1_kernel,
        grid=(N // BR,),
        in_specs=[
            pl.BlockSpec((BR, N), lambda i: (i, 0)),
            pl.BlockSpec((N, D), lambda i: (0, 0)),
            pl.BlockSpec((N, 1), lambda i: (0, 0)),
            pl.BlockSpec((D, H1), lambda i: (0, 0)),
            pl.BlockSpec((H1, H2), lambda i: (0, 0)),
        ],
        out_specs=pl.BlockSpec((BR, H2), lambda i: (i, 0)),
        out_shape=jax.ShapeDtypeStruct((N, H2), jnp.float32),
    )(A, Xp, dinv, W1, W2)


def _layer2_kernel(a_ref, m_ref, dinv_ref, out_ref):
    i = pl.program_id(0)
    a = a_ref[...]
    m = m_ref[...]
    t = jnp.dot(a, m.astype(jnp.bfloat16), preferred_element_type=jnp.float32)
    t = t + m_ref[pl.ds(i * BR, BR), :]
    out_ref[...] = t * dinv_ref[pl.ds(i * BR, BR), :]


def _layer2(A, M, dinv):
    return pl.pallas_call(
        _layer2_kernel,
        grid=(N // BR,),
        in_specs=[
            pl.BlockSpec((BR, N), lambda i: (i, 0)),
            pl.BlockSpec((N, H2), lambda i: (0, 0)),
            pl.BlockSpec((N, 1), lambda i: (0, 0)),
        ],
        out_specs=pl.BlockSpec((BR, H2), lambda i: (i, 0)),
        out_shape=jax.ShapeDtypeStruct((N, H2), jnp.float32),
    )(A, M, dinv)


def _epilogue_kernel(hv_ref, hg_ref, wd1_ref, wd2_ref, wf_ref, bf_ref, c_ref,
                     h_ref, q_ref, p_ref, xhat_ref):
    hv = hv_ref[...]
    hg = hg_ref[...]
    r = jnp.maximum(jnp.dot(hv, wd1_ref[...], preferred_element_type=jnp.float32), 0.0)
    xhat_ref[...] = jnp.dot(r, wd2_ref[...], preferred_element_type=jnp.float32)
    wf = wf_ref[...]
    t = (jnp.dot(hv, wf[:H2], preferred_element_type=jnp.float32)
         + jnp.dot(hg, wf[H2:], preferred_element_type=jnp.float32)
         + bf_ref[...])
    h = jnp.tanh(t)
    h_ref[...] = h
    c = c_ref[...]
    cross = jnp.dot(h, c.T, preferred_element_type=jnp.float32)
    dist2 = (jnp.sum(h * h, axis=1, keepdims=True)
             + jnp.sum(c * c, axis=1)[None, :]
             - 2.0 * cross)
    q = 1.0 / (1.0 + dist2)
    q = q / jnp.sum(q, axis=1, keepdims=True)
    q_ref[...] = q
    f = jnp.sum(q, axis=0, keepdims=True)
    p = (q * q) / f
    p_ref[...] = p / jnp.sum(p, axis=1, keepdims=True)


def _epilogue(h_v, h_g, Wd1, Wd2, Wf, bf, centers):
    return pl.pallas_call(
        _epilogue_kernel,
        out_shape=[
            jax.ShapeDtypeStruct((N, ODIM), jnp.float32),
            jax.ShapeDtypeStruct((N, K), jnp.float32),
            jax.ShapeDtypeStruct((N, K), jnp.float32),
            jax.ShapeDtypeStruct((N, D), jnp.float32),
        ],
    )(h_v, h_g, Wd1, Wd2, Wf, bf, centers)


def kernel(X, adj_v, adj_glo, W1_v, W2_v, W1_g, W2_g, Wd1, Wd2, Wf, bf, centers):
    dinv_v, Xp_v, Abf_v = _rowsum_prep(adj_v, X)
    m2_v = _layer1(Abf_v, Xp_v, dinv_v, W1_v, W2_v)
    h_v = _layer2(Abf_v, m2_v, dinv_v)

    dinv_g, Xp_g, Abf_g = _rowsum_prep(adj_glo, X)
    m2_g = _layer1(Abf_g, Xp_g, dinv_g, W1_g, W2_g)
    h_g = _layer2(Abf_g, m2_g, dinv_g)

    h, q, p, X_hat = _epilogue(h_v, h_g, Wd1, Wd2, Wf, bf.reshape(1, ODIM), centers)
    return (h, q, p, X_hat)


# speedup vs baseline: 1.2882x; 1.2472x over previous
import jax
import jax.numpy as jnp
from jax.experimental import pallas as pl
from jax.experimental.pallas import tpu as pltpu

N = 4096
D = 128
H1 = 256
H2 = 64
ODIM = 32
K = 10
BRC = 256
NB = N // BRC


def _branch_kernel(a_ref, x_ref, w1_ref, w2_ref, h_ref,
                   abf, xpf, xpb, dinv, m2f, m2b):
    p = pl.program_id(0)
    i = pl.program_id(1)
    rows = pl.ds(i * BRC, BRC)

    @pl.when(p == 0)
    def _():
        a = a_ref[...]
        d = jnp.sum(a, axis=1, keepdims=True) + 1.0
        dv = jax.lax.rsqrt(jnp.maximum(d, 1e-12))
        dinv[rows, :] = dv
        xp = x_ref[...] * dv
        xpf[rows, :] = xp
        xpb[rows, :] = xp.astype(jnp.bfloat16)
        abf[rows, :] = a.astype(jnp.bfloat16)

    @pl.when(p == 1)
    def _():
        a = abf[rows, :]
        t = jnp.dot(a, xpb[...], preferred_element_type=jnp.float32)
        dv = dinv[rows, :]
        t = (t + xpf[rows, :]) * dv
        r = jnp.maximum(jnp.dot(t, w1_ref[...], preferred_element_type=jnp.float32), 0.0)
        m2 = jnp.dot(r, w2_ref[...], preferred_element_type=jnp.float32) * dv
        m2f[rows, :] = m2
        m2b[rows, :] = m2.astype(jnp.bfloat16)

    @pl.when(p == 2)
    def _():
        a = abf[rows, :]
        t = jnp.dot(a, m2b[...], preferred_element_type=jnp.float32)
        h_ref[...] = (t + m2f[rows, :]) * dinv[rows, :]


def _branch(A, X, W1, W2):
    return pl.pallas_call(
        _branch_kernel,
        grid=(3, NB),
        in_specs=[
            pl.BlockSpec((BRC, N), lambda p, i: (jnp.where(p == 0, i, 0), 0)),
            pl.BlockSpec((BRC, D), lambda p, i: (jnp.where(p == 0, i, 0), 0)),
            pl.BlockSpec((D, H1), lambda p, i: (0, 0)),
            pl.BlockSpec((H1, H2), lambda p, i: (0, 0)),
        ],
        out_specs=pl.BlockSpec((BRC, H2), lambda p, i: (i, 0)),
        out_shape=jax.ShapeDtypeStruct((N, H2), jnp.float32),
        scratch_shapes=[
            pltpu.VMEM((N, N), jnp.bfloat16),
            pltpu.VMEM((N, D), jnp.float32),
            pltpu.VMEM((N, D), jnp.bfloat16),
            pltpu.VMEM((N, 1), jnp.float32),
            pltpu.VMEM((N, H2), jnp.float32),
            pltpu.VMEM((N, H2), jnp.bfloat16),
        ],
    )(A, X, W1, W2)


def _epilogue_kernel(hv_ref, hg_ref, wd1_ref, wd2_ref, wf_ref, bf_ref, c_ref,
                     h_ref, q_ref, p_ref, xhat_ref):
    hv = hv_ref[...]
    hg = hg_ref[...]
    r = jnp.maximum(jnp.dot(hv, wd1_ref[...], preferred_element_type=jnp.float32), 0.0)
    xhat_ref[...] = jnp.dot(r, wd2_ref[...], preferred_element_type=jnp.float32)
    wf = wf_ref[...]
    t = (jnp.dot(hv, wf[:H2], preferred_element_type=jnp.float32)
         + jnp.dot(hg, wf[H2:], preferred_element_type=jnp.float32)
         + bf_ref[...])
    h = jnp.tanh(t)
    h_ref[...] = h
    c = c_ref[...]
    cross = jnp.dot(h, c.T, preferred_element_type=jnp.float32)
    dist2 = (jnp.sum(h * h, axis=1, keepdims=True)
             + jnp.sum(c * c, axis=1)[None, :]
             - 2.0 * cross)
    q = 1.0 / (1.0 + dist2)
    q = q / jnp.sum(q, axis=1, keepdims=True)
    q_ref[...] = q
    f = jnp.sum(q, axis=0, keepdims=True)
    p = (q * q) / f
    p_ref[...] = p / jnp.sum(p, axis=1, keepdims=True)


def _epilogue(h_v, h_g, Wd1, Wd2, Wf, bf, centers):
    return pl.pallas_call(
        _epilogue_kernel,
        out_shape=[
            jax.ShapeDtypeStruct((N, ODIM), jnp.float32),
            jax.ShapeDtypeStruct((N, K), jnp.float32),
            jax.ShapeDtypeStruct((N, K), jnp.float32),
            jax.ShapeDtypeStruct((N, D), jnp.float32),
        ],
    )(h_v, h_g, Wd1, Wd2, Wf, bf, centers)


def kernel(X, adj_v, adj_glo, W1_v, W2_v, W1_g, W2_g, Wd1, Wd2, Wf, bf, centers):
    h_v = _branch(adj_v, X, W1_v, W2_v)
    h_g = _branch(adj_glo, X, W1_g, W2_g)
    h, q, p, X_hat = _epilogue(h_v, h_g, Wd1, Wd2, Wf, bf.reshape(1, ODIM), centers)
    return (h, q, p, X_hat)


# speedup vs baseline: 1.4190x; 1.1016x over previous
import jax
import jax.numpy as jnp
from jax.experimental import pallas as pl
from jax.experimental.pallas import tpu as pltpu

N = 4096
D = 128
H1 = 256
H2 = 64
ODIM = 32
K = 10
BRC = 512
NB = N // BRC


def _branch_kernel(a_ref, x_ref, w1_ref, w2_ref, h_ref,
                   abf, xpb, dinv, m2b):
    p = pl.program_id(0)
    i = pl.program_id(1)
    rows = pl.ds(i * BRC, BRC)

    @pl.when(p == 0)
    def _():
        a = a_ref[...].astype(jnp.bfloat16)
        abf[rows, :] = a
        ones = jnp.ones((N, 8), jnp.bfloat16)
        d = jnp.dot(a, ones, preferred_element_type=jnp.float32)[:, :1] + 1.0
        dv = jax.lax.rsqrt(jnp.maximum(d, 1e-12))
        dinv[rows, :] = dv
        xpb[rows, :] = (x_ref[...] * dv).astype(jnp.bfloat16)

    @pl.when(p == 1)
    def _():
        a = abf[rows, :]
        t = jnp.dot(a, xpb[...], preferred_element_type=jnp.float32)
        dv = dinv[rows, :]
        t = (t + xpb[rows, :]) * dv
        r = jnp.maximum(jnp.dot(t, w1_ref[...], preferred_element_type=jnp.float32), 0.0)
        m2 = jnp.dot(r, w2_ref[...], preferred_element_type=jnp.float32) * dv
        m2b[rows, :] = m2.astype(jnp.bfloat16)

    @pl.when(p == 2)
    def _():
        a = abf[rows, :]
        t = jnp.dot(a, m2b[...], preferred_element_type=jnp.float32)
        h_ref[...] = (t + m2b[rows, :]) * dinv[rows, :]


def _branch(A, X, W1, W2):
    return pl.pallas_call(
        _branch_kernel,
        grid=(3, NB),
        in_specs=[
            pl.BlockSpec((BRC, N), lambda p, i: (jnp.where(p == 0, i, NB - 1), 0)),
            pl.BlockSpec((BRC, D), lambda p, i: (jnp.where(p == 0, i, NB - 1), 0)),
            pl.BlockSpec((D, H1), lambda p, i: (0, 0)),
            pl.BlockSpec((H1, H2), lambda p, i: (0, 0)),
        ],
        out_specs=pl.BlockSpec((BRC, H2), lambda p, i: (i, 0)),
        out_shape=jax.ShapeDtypeStruct((N, H2), jnp.float32),
        scratch_shapes=[
            pltpu.VMEM((N, N), jnp.bfloat16),
            pltpu.VMEM((N, D), jnp.bfloat16),
            pltpu.VMEM((N, 1), jnp.float32),
            pltpu.VMEM((N, H2), jnp.bfloat16),
        ],
    )(A, X, W1, W2)


def _epilogue_kernel(hv_ref, hg_ref, wd1_ref, wd2_ref, wf_ref, bf_ref, c_ref,
                     h_ref, q_ref, p_ref, xhat_ref):
    hv = hv_ref[...]
    hg = hg_ref[...]
    r = jnp.maximum(jnp.dot(hv, wd1_ref[...], preferred_element_type=jnp.float32), 0.0)
    xhat_ref[...] = jnp.dot(r, wd2_ref[...], preferred_element_type=jnp.float32)
    wf = wf_ref[...]
    t = (jnp.dot(hv, wf[:H2], preferred_element_type=jnp.float32)
         + jnp.dot(hg, wf[H2:], preferred_element_type=jnp.float32)
         + bf_ref[...])
    h = jnp.tanh(t)
    h_ref[...] = h
    c = c_ref[...]
    cross = jnp.dot(h, c.T, preferred_element_type=jnp.float32)
    dist2 = (jnp.sum(h * h, axis=1, keepdims=True)
             + jnp.sum(c * c, axis=1)[None, :]
             - 2.0 * cross)
    q = 1.0 / (1.0 + dist2)
    q = q / jnp.sum(q, axis=1, keepdims=True)
    q_ref[...] = q
    f = jnp.sum(q, axis=0, keepdims=True)
    p = (q * q) / f
    p_ref[...] = p / jnp.sum(p, axis=1, keepdims=True)


def _epilogue(h_v, h_g, Wd1, Wd2, Wf, bf, centers):
    return pl.pallas_call(
        _epilogue_kernel,
        out_shape=[
            jax.ShapeDtypeStruct((N, ODIM), jnp.float32),
            jax.ShapeDtypeStruct((N, K), jnp.float32),
            jax.ShapeDtypeStruct((N, K), jnp.float32),
            jax.ShapeDtypeStruct((N, D), jnp.float32),
        ],
    )(h_v, h_g, Wd1, Wd2, Wf, bf, centers)


def kernel(X, adj_v, adj_glo, W1_v, W2_v, W1_g, W2_g, Wd1, Wd2, Wf, bf, centers):
    h_v = _branch(adj_v, X, W1_v, W2_v)
    h_g = _branch(adj_glo, X, W1_g, W2_g)
    h, q, p, X_hat = _epilogue(h_v, h_g, Wd1, Wd2, Wf, bf.reshape(1, ODIM), centers)
    return (h, q, p, X_hat)
